# Initial kernel scaffold; baseline (speedup 1.0000x reference)
import functools

import jax
import jax.numpy as jnp
from jax import lax
from jax.experimental import pallas as pl
from jax.experimental.pallas import tpu as pltpu
from jax.experimental.pallas import tpu_sc as plsc

V = 1000000
D = 64
B = 16384
C = 20
NEG = 20
TN = NEG + 1

NUM_WORKERS = 32
PER_W = B // NUM_WORKERS
E = 16
CHUNKS = PER_W // E
L = 16


def _body(ctx_idx_hbm, tn_idx_hbm, iw_hbm, ow_hbm, out_hbm,
          ctx_idx_v, tn_idx_v, ctx_rows, tn_rows, scores_v, sem):
  wid = lax.axis_index("s") * 2 + lax.axis_index("c")
  ev = lax.iota(jnp.int32, L)
  inv_c = jnp.float32(1.0 / C)

  @pl.loop(0, CHUNKS)
  def _chunk(c):
    base = wid * PER_W + c * E

    pltpu.sync_copy(ctx_idx_hbm.at[pl.ds(base * C, E * C)], ctx_idx_v)
    pltpu.sync_copy(tn_idx_hbm.at[pl.ds(base * TN, E * TN)], tn_idx_v)

    cp1 = pltpu.async_copy(iw_hbm.at[ctx_idx_v], ctx_rows, sem)
    cp2 = pltpu.async_copy(ow_hbm.at[tn_idx_v], tn_rows, sem)
    cp1.wait()
    cp2.wait()

    ctx_row0 = ev * C
    tn_row0 = ev * TN

    def dim_step(d, acc):
      dcol = jnp.full((L,), d, dtype=jnp.int32)
      s = plsc.load_gather(ctx_rows, [ctx_row0, dcol])
      for r in range(1, C):
        s = s + plsc.load_gather(ctx_rows, [ctx_row0 + r, dcol])
      avg = s * inv_c
      return tuple(
          acc[j] + avg * plsc.load_gather(tn_rows, [tn_row0 + j, dcol])
          for j in range(TN))

    zeros = jnp.zeros((L,), jnp.float32)
    acc = lax.fori_loop(0, D, dim_step, (zeros,) * TN, unroll=False)

    for j in range(TN):
      plsc.store_scatter(scores_v, [ev, jnp.full((L,), j, jnp.int32)], acc[j])

    pltpu.sync_copy(scores_v, out_hbm.at[pl.ds(base, E)])


def _cbow_scores(ctx_idx, tn_idx, input_weight, output_weight):
  mesh = plsc.VectorSubcoreMesh(core_axis_name="c", subcore_axis_name="s")
  f = pl.kernel(
      _body,
      out_type=jax.ShapeDtypeStruct((B, TN), jnp.float32),
      mesh=mesh,
      scratch_types=[
          pltpu.VMEM((E * C,), jnp.int32),
          pltpu.VMEM((E * TN,), jnp.int32),
          pltpu.VMEM((E * C, D), jnp.float32),
          pltpu.VMEM((E * TN, D), jnp.float32),
          pltpu.VMEM((E, TN), jnp.float32),
          pltpu.SemaphoreType.DMA,
      ],
  )
  return f(ctx_idx, tn_idx, input_weight, output_weight)


def kernel(context_words, target_word, negative_samples, input_weight,
           output_weight):
  ctx_idx = context_words.astype(jnp.int32).reshape(-1)
  tn_idx = jnp.concatenate(
      [target_word.astype(jnp.int32)[:, None],
       negative_samples.astype(jnp.int32)], axis=1).reshape(-1)
  scores = _cbow_scores(ctx_idx, tn_idx, input_weight, output_weight)
  return scores[:, 0], scores[:, 1:]


# speedup vs baseline: 2.7378x; 2.7378x over previous
import jax
import jax.numpy as jnp
from jax import lax
from jax.experimental import pallas as pl
from jax.experimental.pallas import tpu as pltpu
from jax.experimental.pallas import tpu_sc as plsc

V = 1000000
D = 64
B = 16384
C = 20
NEG = 20
TN = NEG + 1
W = D // 2

NUM_WORKERS = 32
PER_W = B // NUM_WORKERS
E = 16
CHUNKS = PER_W // E
L = 16

def _pack_table(table):
  lo = table[:, :W].astype(jnp.bfloat16)
  hi = table[:, W:].astype(jnp.bfloat16)
  lo32 = lax.bitcast_convert_type(lo, jnp.uint16).astype(jnp.uint32)
  hi32 = lax.bitcast_convert_type(hi, jnp.uint16).astype(jnp.uint32)
  return lax.bitcast_convert_type(lo32 | (hi32 << 16), jnp.int32)


def _body(ctx_idx_hbm, tn_idx_hbm, iw_hbm, ow_hbm, out_hbm,
          ctx_idx_v, tn_idx_v, crows0, trows0, crows1, trows1,
          scores_v, sem0, sem1):
  wid = lax.axis_index("s") * 2 + lax.axis_index("c")
  wbase = wid * PER_W
  ev = lax.iota(jnp.int32, L)
  inv_c = jnp.float32(1.0 / C)
  evC = ev * C
  evTN = ev * TN
  himask = jnp.full((L,), -65536, jnp.int32)

  pltpu.sync_copy(ctx_idx_hbm.at[pl.ds(wbase * C, PER_W * C)], ctx_idx_v)
  pltpu.sync_copy(tn_idx_hbm.at[pl.ds(wbase * TN, PER_W * TN)], tn_idx_v)

  bufs = ((crows0, trows0, sem0), (crows1, trows1, sem1))

  def fire(c, crows, trows, sem):
    pltpu.async_copy(
        iw_hbm.at[ctx_idx_v.at[pl.ds(c * (E * C), E * C)]], crows, sem)
    pltpu.async_copy(
        ow_hbm.at[tn_idx_v.at[pl.ds(c * (E * TN), E * TN)]], trows, sem)

  def drain(crows, trows, sem):
    pltpu.make_async_copy(
        iw_hbm.at[ctx_idx_v.at[pl.ds(0, E * C)]], crows, sem).wait()
    pltpu.make_async_copy(
        ow_hbm.at[tn_idx_v.at[pl.ds(0, E * TN)]], trows, sem).wait()

  def unpack(word):
    f_lo = plsc.bitcast(word << 16, jnp.float32)
    f_hi = plsc.bitcast(word & himask, jnp.float32)
    return f_lo, f_hi

  def compute(c, crows, trows):
    cev = c * E + ev

    def word_scores(k):
      kcol = jnp.full((L,), k, jnp.int32)
      s_lo, s_hi = unpack(plsc.load_gather(crows, [evC, kcol]))
      for r in range(1, C):
        lo, hi = unpack(plsc.load_gather(crows, [evC + r, kcol]))
        s_lo = s_lo + lo
        s_hi = s_hi + hi
      a_lo = s_lo * inv_c
      a_hi = s_hi * inv_c
      out = []
      for j in range(TN):
        lo, hi = unpack(plsc.load_gather(trows, [evTN + j, kcol]))
        out.append(a_lo * lo + a_hi * hi)
      return out

    sc0 = word_scores(0)
    for j in range(TN):
      plsc.store_scatter(
          scores_v, [cev, jnp.full((L,), j, jnp.int32)], sc0[j])

    @pl.loop(1, W)
    def _words(k):
      sck = word_scores(k)
      for j in range(TN):
        plsc.addupdate_scatter(
            scores_v, [cev, jnp.full((L,), j, jnp.int32)], sck[j])

  fire(0, *bufs[0])

  @pl.loop(0, CHUNKS, step=2)
  def _chunks(c):
    fire(c + 1, *bufs[1])
    drain(*bufs[0])
    compute(c, bufs[0][0], bufs[0][1])

    @pl.when(c + 2 < CHUNKS)
    def _prefetch():
      fire(c + 2, *bufs[0])

    drain(*bufs[1])
    compute(c + 1, bufs[1][0], bufs[1][1])

  pltpu.sync_copy(scores_v, out_hbm.at[pl.ds(wbase, PER_W)])


def _cbow_scores(ctx_idx, tn_idx, iw_packed, ow_packed):
  mesh = plsc.VectorSubcoreMesh(core_axis_name="c", subcore_axis_name="s")
  f = pl.kernel(
      _body,
      out_type=jax.ShapeDtypeStruct((B, TN), jnp.float32),
      mesh=mesh,
      scratch_types=[
          pltpu.VMEM((PER_W * C,), jnp.int32),
          pltpu.VMEM((PER_W * TN,), jnp.int32),
          pltpu.VMEM((E * C, W), jnp.int32),
          pltpu.VMEM((E * TN, W), jnp.int32),
          pltpu.VMEM((E * C, W), jnp.int32),
          pltpu.VMEM((E * TN, W), jnp.int32),
          pltpu.VMEM((PER_W, TN), jnp.float32),
          pltpu.SemaphoreType.DMA,
          pltpu.SemaphoreType.DMA,
      ],
      compiler_params=pltpu.CompilerParams(
          needs_layout_passes=False, use_tc_tiling_on_sc=False),
  )
  return f(ctx_idx, tn_idx, iw_packed, ow_packed)


def kernel(context_words, target_word, negative_samples, input_weight,
           output_weight):
  ctx_idx = context_words.astype(jnp.int32).reshape(-1)
  tn_idx = jnp.concatenate(
      [target_word.astype(jnp.int32)[:, None],
       negative_samples.astype(jnp.int32)], axis=1).reshape(-1)
  iw_packed = _pack_table(input_weight)
  ow_packed = _pack_table(output_weight)
  scores = _cbow_scores(ctx_idx, tn_idx, iw_packed, ow_packed)
  return scores[:, 0], scores[:, 1:]
